# baseline (speedup 1.0000x reference)
import jax
import jax.numpy as jnp
from jax import lax
from jax.experimental import pallas as pl
from jax.experimental.pallas import tpu as pltpu
from jax.experimental.pallas import tpu_sc as plsc

N_VAR = 4
N_POINTS = 49152
D = 128
B = 2
V = 2
P = 4096

NC = 2
NS = 16
NW = NC * NS
ROWS_PER_W = (B * V * P) // NW
CH = 128
NCH = ROWS_PER_W // CH


def _gather_body(table_hbm, var_hbm, patch_hbm, out_hbm, *scr):
    idxs = scr[:NCH]
    rows_v = scr[NCH]
    var_v = scr[NCH + 1]
    gsem = scr[NCH + 2:2 * NCH + 2]
    sem_w = scr[2 * NCH + 2]
    c = lax.axis_index("c")
    s = lax.axis_index("s")
    w = s * NC + c
    pair = w // 8
    b = pair // V
    v = pair % V
    chunk = w % 8
    pbase = chunk * ROWS_PER_W

    stages = [
        pltpu.async_copy(patch_hbm.at[b, pl.ds(pbase + j * CH, CH)], idxs[j],
                         gsem[j])
        for j in range(NCH)
    ]
    pltpu.sync_copy(var_hbm.at[pair], var_v)

    off = var_v[...] * N_POINTS

    half = NCH // 2

    def fire_half(h):
        return pltpu.async_copy(
            rows_v.at[pl.ds(h * half, half)],
            out_hbm.at[b, v, 0, pl.ds(chunk * NCH + h * half, half)], sem_w)

    gathers = []
    for j in range(NCH):
        stages[j].wait()
        for i in range(CH // 16):
            sl = pl.ds(i * 16, 16)
            idxs[j][sl] = idxs[j][sl] + off
        gathers.append(
            pltpu.async_copy(table_hbm.at[idxs[j]], rows_v.at[j], gsem[j]))
        if j == NCH - 2:
            for g in gathers[:half]:
                g.wait()
            wb0 = fire_half(0)
    for g in gathers[half:]:
        g.wait()
    wb1 = fire_half(1)
    wb0.wait()
    wb1.wait()


def kernel(mg_embedding, var_indices, patch_idx):
    table2d = mg_embedding.reshape(N_VAR * N_POINTS, D)
    var_tab = jnp.broadcast_to(
        var_indices.astype(jnp.int32).reshape(B * V, 1), (B * V, 16))

    run = pl.kernel(
        _gather_body,
        out_type=jax.ShapeDtypeStruct((B, V, 1, P // CH, CH, D), jnp.float32),
        mesh=plsc.VectorSubcoreMesh(core_axis_name="c", subcore_axis_name="s"),
        scratch_types=(
            [pltpu.VMEM((CH,), jnp.int32) for _ in range(NCH)]
            + [pltpu.VMEM((NCH, CH, D), jnp.float32)]
            + [pltpu.VMEM((16,), jnp.int32)]
            + [pltpu.SemaphoreType.DMA for _ in range(NCH + 1)]
        ),
    )
    out = run(table2d, var_tab, patch_idx.astype(jnp.int32))
    return out.reshape(B, V, 1, P, D)


# speedup vs baseline: 1.0227x; 1.0227x over previous
import jax
import jax.numpy as jnp
from jax import lax
from jax.experimental import pallas as pl
from jax.experimental.pallas import tpu as pltpu
from jax.experimental.pallas import tpu_sc as plsc

N_VAR = 4
N_POINTS = 49152
D = 128
B = 2
V = 2
P = 4096

NC = 2
NS = 16
NW = NC * NS
ROWS_PER_W = (B * V * P) // NW
CH = 128
NCH = ROWS_PER_W // CH


def _gather_body(table_hbm, var_hbm, patch_hbm, out_hbm, *scr):
    idxs = scr[:NCH]
    rows_v = scr[NCH]
    var_v = scr[NCH + 1]
    sbuf = scr[NCH + 2]
    gsem = scr[NCH + 3:2 * NCH + 3]
    c = lax.axis_index("c")
    s = lax.axis_index("s")
    w = s * NC + c
    pair = w // 8
    b = pair // V
    v = pair % V
    chunk = w % 8
    pbase = chunk * ROWS_PER_W

    stage = pltpu.async_copy(
        patch_hbm.at[b, pl.ds(pbase, ROWS_PER_W)], sbuf, gsem[0])
    pltpu.sync_copy(var_hbm.at[pair], var_v)

    off = var_v[...] * N_POINTS

    stage.wait()
    gathers = []
    for j in range(NCH):
        for i in range(CH // 16):
            idxs[j][pl.ds(i * 16, 16)] = (
                sbuf[pl.ds(j * CH + i * 16, 16)] + off)
        gathers.append(
            pltpu.async_copy(table_hbm.at[idxs[j]], rows_v.at[j], gsem[j]))
    for g in gathers:
        g.wait()

    pltpu.sync_copy(rows_v, out_hbm.at[b, v, 0, pl.ds(chunk * NCH, NCH)])


def kernel(mg_embedding, var_indices, patch_idx):
    table2d = mg_embedding.reshape(N_VAR * N_POINTS, D)
    var_tab = jnp.broadcast_to(
        var_indices.astype(jnp.int32).reshape(B * V, 1), (B * V, 16))

    run = pl.kernel(
        _gather_body,
        out_type=jax.ShapeDtypeStruct((B, V, 1, P // CH, CH, D), jnp.float32),
        mesh=plsc.VectorSubcoreMesh(core_axis_name="c", subcore_axis_name="s"),
        scratch_types=(
            [pltpu.VMEM((CH,), jnp.int32) for _ in range(NCH)]
            + [pltpu.VMEM((NCH, CH, D), jnp.float32)]
            + [pltpu.VMEM((16,), jnp.int32)]
            + [pltpu.VMEM((ROWS_PER_W,), jnp.int32)]
            + [pltpu.SemaphoreType.DMA for _ in range(NCH)]
        ),
    )
    out = run(table2d, var_tab, patch_idx.astype(jnp.int32))
    return out.reshape(B, V, 1, P, D)


# speedup vs baseline: 1.0394x; 1.0163x over previous
import jax
import jax.numpy as jnp
from jax import lax
from jax.experimental import pallas as pl
from jax.experimental.pallas import tpu as pltpu
from jax.experimental.pallas import tpu_sc as plsc

N_VAR = 4
N_POINTS = 49152
D = 128
B = 2
V = 2
P = 4096

NC = 2
NS = 16
NW = NC * NS
ROWS_PER_W = (B * V * P) // NW
CH = 128
NCH = ROWS_PER_W // CH


def _gather_body(table_hbm, var_hbm, patch_hbm, out_hbm, *scr):
    idx_v = scr[0]
    rows_v = scr[1]
    var_v = scr[2]
    sbuf = scr[3]
    gsem = scr[4]
    c = lax.axis_index("c")
    s = lax.axis_index("s")
    w = s * NC + c
    pair = w // 8
    b = pair // V
    v = pair % V
    chunk = w % 8
    pbase = chunk * ROWS_PER_W

    stage = pltpu.async_copy(
        patch_hbm.at[b, pl.ds(pbase, ROWS_PER_W)], sbuf, gsem)
    pltpu.sync_copy(var_hbm.at[pair], var_v)

    off = var_v[...] * N_POINTS

    stage.wait()
    for i in range(ROWS_PER_W // 16):
        idx_v[pl.ds(i * 16, 16)] = sbuf[pl.ds(i * 16, 16)] + off
    pltpu.async_copy(table_hbm.at[idx_v], rows_v, gsem).wait()

    pltpu.sync_copy(rows_v, out_hbm.at[b, v, 0, pl.ds(pbase, ROWS_PER_W), :])


def kernel(mg_embedding, var_indices, patch_idx):
    table2d = mg_embedding.reshape(N_VAR * N_POINTS, D)
    var_tab = jnp.broadcast_to(
        var_indices.astype(jnp.int32).reshape(B * V, 1), (B * V, 16))

    run = pl.kernel(
        _gather_body,
        out_type=jax.ShapeDtypeStruct((B, V, 1, P, D), jnp.float32),
        mesh=plsc.VectorSubcoreMesh(core_axis_name="c", subcore_axis_name="s"),
        scratch_types=(
            [pltpu.VMEM((ROWS_PER_W,), jnp.int32)]
            + [pltpu.VMEM((ROWS_PER_W, D), jnp.float32)]
            + [pltpu.VMEM((16,), jnp.int32)]
            + [pltpu.VMEM((ROWS_PER_W,), jnp.int32)]
            + [pltpu.SemaphoreType.DMA]
        ),
    )
    return run(table2d, var_tab, patch_idx.astype(jnp.int32))
